# baseline (speedup 1.0000x reference)
import jax
import jax.numpy as jnp
from jax.experimental import pallas as pl
from jax.experimental.pallas import tpu as pltpu

B, G, D = 16, 20000, 128
GB = 256


def _body(x_ref, w_ref, b_ref, o_ref):
    x = x_ref[...]
    w = w_ref[...]
    bb = b_ref[...]
    o_ref[...] = jnp.maximum(x[:, :, None] * w[None] + bb[None], 0.0)


def kernel(x, weight, bias):
    return pl.pallas_call(
        _body,
        grid=(pl.cdiv(G, GB),),
        in_specs=[
            pl.BlockSpec((B, GB), lambda i: (0, i)),
            pl.BlockSpec((GB, D), lambda i: (i, 0)),
            pl.BlockSpec((GB, D), lambda i: (i, 0)),
        ],
        out_specs=pl.BlockSpec((B, GB, D), lambda i: (0, i, 0)),
        out_shape=jax.ShapeDtypeStruct((B, G, D), jnp.float32),
        compiler_params=pltpu.CompilerParams(
            dimension_semantics=("arbitrary",),
        ),
    )(x, weight, bias)


# speedup vs baseline: 1.5401x; 1.5401x over previous
import jax
import jax.numpy as jnp
from jax.experimental import pallas as pl
from jax.experimental.pallas import tpu as pltpu

B, G, D = 16, 20000, 128
GB = 2048


def _body(x_ref, w_ref, b_ref, o_ref):
    x = x_ref[...]
    w = w_ref[...]
    bb = b_ref[...]
    o_ref[...] = jnp.maximum(x[:, :, None] * w[None] + bb[None], 0.0)


def kernel(x, weight, bias):
    return pl.pallas_call(
        _body,
        grid=(pl.cdiv(G, GB),),
        in_specs=[
            pl.BlockSpec((B, GB), lambda i: (0, i)),
            pl.BlockSpec((GB, D), lambda i: (i, 0)),
            pl.BlockSpec((GB, D), lambda i: (i, 0)),
        ],
        out_specs=pl.BlockSpec((B, GB, D), lambda i: (0, i, 0)),
        out_shape=jax.ShapeDtypeStruct((B, G, D), jnp.float32),
        compiler_params=pltpu.CompilerParams(
            dimension_semantics=("arbitrary",),
        ),
    )(x, weight, bias)
